# baseline (speedup 1.0000x reference)
import functools

import jax
import jax.numpy as jnp
from jax import lax
from jax.experimental import pallas as pl
from jax.experimental.pallas import tpu as pltpu
from jax.experimental.pallas import tpu_sc as plsc

NC = 2
NS = 16
NW = NC * NS
LANES = 16


def _sc_segment_sum(h, src, dst, e=None, chunk=80, nbuf=4):
    N, D = h.shape
    E = src.shape[0]
    EPW = E // NW
    n_chunks = -(-EPW // chunk)
    n_cpad = 2 * (-(-n_chunks // 2))
    WC = 80 if chunk >= 80 else 40
    NCH = N // WC
    npt = -(-NCH // NS)
    vecs = D // LANES
    fuse = e is not None
    if fuse:
        assert EPW % chunk == 0

    EPP = n_cpad * chunk
    pad = EPP - EPW
    src3 = src.reshape(NW, EPW)
    dst3 = dst.reshape(NW, EPW)
    if pad:
        src3 = jnp.pad(src3, ((0, 0), (0, pad)))
        dst3 = jnp.pad(dst3, ((0, 0), (0, pad)), constant_values=N)
    src3 = src3.reshape(NW * EPP)
    dst3 = dst3.reshape(NW * EPP)

    mesh = plsc.VectorSubcoreMesh(core_axis_name="c", subcore_axis_name="s")

    scratch = [
        pltpu.VMEM_SHARED((N + 8, D), jnp.float32),
        [pltpu.VMEM((chunk,), jnp.int32) for _ in range(2)],
        [pltpu.VMEM((chunk,), jnp.int32) for _ in range(2)],
        [pltpu.VMEM((chunk, D), jnp.float32) for _ in range(2)],
        [pltpu.SemaphoreType.DMA for _ in range(2)],
        [pltpu.SemaphoreType.DMA for _ in range(2)],
        [pltpu.SemaphoreType.DMA for _ in range(2)],
        [pltpu.SemaphoreType.DMA for _ in range(2)],
    ]
    if fuse:
        scratch.append([pltpu.VMEM((chunk, D), jnp.float32) for _ in range(2)])
        scratch.append([pltpu.SemaphoreType.DMA for _ in range(2)])

    def body(*refs):
        if fuse:
            (h_hbm, src_hbm, dst_hbm, e_hbm, out_hbm, acc, srcv, dstv,
             bufs, issems, idsems, gsems, ssems, ebufs, esems) = refs
        else:
            (h_hbm, src_hbm, dst_hbm, out_hbm, acc, srcv, dstv,
             bufs, issems, idsems, gsems, ssems) = refs

        c = lax.axis_index("c")
        s = lax.axis_index("s")
        wid = s * NC + c

        zero = jnp.zeros((LANES,), jnp.float32)

        def zrow(r, _):
            for cc in range(vecs):
                bufs[0][r, pl.ds(cc * LANES, LANES)] = zero
            return 0

        lax.fori_loop(0, WC, zrow, 0, unroll=False)

        for j in range(npt):
            k = s + j * NS

            @pl.when(k < NCH)
            def _():
                pltpu.sync_copy(bufs[0].at[pl.ds(0, WC), :],
                                acc.at[pl.ds(k * WC, WC), :])

        plsc.subcore_barrier()

        def chunkfn(k, _):
            ebase = wid * EPP + k * chunk
            pltpu.sync_copy(src_hbm.at[pl.ds(ebase, chunk)], srcv[0])
            pltpu.sync_copy(dst_hbm.at[pl.ds(ebase, chunk)], dstv[0])
            pltpu.sync_copy(h_hbm.at[srcv[0]], bufs[0])
            if fuse:
                kc = jnp.minimum(k, n_chunks - 1)
                base = wid * EPW + kc * chunk
                pltpu.sync_copy(e_hbm.at[pl.ds(base, chunk), :], ebufs[0])

                def rowfn(r, _):
                    for cc in range(vecs):
                        sl = pl.ds(cc * LANES, LANES)
                        v = bufs[0][r, sl] + ebufs[0][r, sl]
                        bufs[0][r, sl] = jnp.maximum(v, 0.0)
                    return 0

                lax.fori_loop(0, chunk, rowfn, 0, unroll=4)
            pltpu.sync_copy(bufs[0], acc.at[dstv[0]], add=True)
            return 0

        lax.fori_loop(0, n_cpad, chunkfn, 0, unroll=False)
        plsc.subcore_barrier()

        for j in range(npt):
            k = s + j * NS

            @pl.when(k < NCH)
            def _():
                pltpu.sync_copy(acc.at[pl.ds(k * WC, WC), :],
                                bufs[0].at[pl.ds(0, WC), :])
                pltpu.sync_copy(bufs[0].at[pl.ds(0, WC), :],
                                out_hbm.at[c, pl.ds(k * WC, WC), :])

    kern = pl.kernel(
        body,
        out_type=jax.ShapeDtypeStruct((2, N, D), jnp.float32),
        mesh=mesh,
        scratch_types=scratch,
    )
    if fuse:
        return kern(h, src3, dst3, e)
    return kern(h, src3, dst3)


def _tc_edge_mlp(edge_attr, W_edge, b_edge):
    E, DE = edge_attr.shape
    D = W_edge.shape[1]
    BE = 4000

    def body(ea_ref, w_ref, b_ref, out_ref):
        out_ref[...] = jnp.dot(ea_ref[...], w_ref[...],
                               preferred_element_type=jnp.float32) + b_ref[...]

    return pl.pallas_call(
        body,
        grid=(E // BE,),
        in_specs=[
            pl.BlockSpec((BE, DE), lambda i: (i, 0)),
            pl.BlockSpec((DE, D), lambda i: (0, 0)),
            pl.BlockSpec((1, D), lambda i: (0, 0)),
        ],
        out_specs=pl.BlockSpec((BE, D), lambda i: (i, 0)),
        out_shape=jax.ShapeDtypeStruct((E, D), jnp.float32),
    )(edge_attr, W_edge, b_edge.reshape(1, D))


def _bn_relu(t, g, be):
    m = jnp.mean(t, axis=0, keepdims=True)
    d = t - m
    v = jnp.mean(d * d, axis=0, keepdims=True)
    return jnp.maximum(g * d * lax.rsqrt(v + 1e-5) + be, 0.0)


def _tc_stage1(x, parts, W, b, g, be):
    N, D = x.shape

    def body(x_ref, p_ref, w_ref, b_ref, g_ref, be_ref, out_ref):
        a = x_ref[...] + p_ref[0] + p_ref[1]
        t = jnp.dot(a, w_ref[...], preferred_element_type=jnp.float32)
        t = t + b_ref[...]
        out_ref[...] = _bn_relu(t, g_ref[...], be_ref[...])

    return pl.pallas_call(
        body,
        out_shape=jax.ShapeDtypeStruct((N, D), jnp.float32),
    )(x, parts, W, b.reshape(1, D), g.reshape(1, D), be.reshape(1, D))


def _tc_stage23(h, parts, Wl, Wr, b, g, be):
    N, D = h.shape

    def body(h_ref, p_ref, wl_ref, wr_ref, b_ref, g_ref, be_ref, out_ref):
        t = jnp.dot(h_ref[...], wl_ref[...], preferred_element_type=jnp.float32)
        t = t + jnp.dot(p_ref[0] + p_ref[1], wr_ref[...],
                        preferred_element_type=jnp.float32)
        t = t + b_ref[...]
        out_ref[...] = _bn_relu(t, g_ref[...], be_ref[...])

    return pl.pallas_call(
        body,
        out_shape=jax.ShapeDtypeStruct((N, D), jnp.float32),
    )(h, parts, Wl, Wr, b.reshape(1, D), g.reshape(1, D), be.reshape(1, D))


def _tc_final(h, parts, Wl, Wr, b, g, be, batch, Wc, bc, G):
    N, D = h.shape
    KC = 8
    Wc_p = jnp.zeros((D, KC), jnp.float32).at[:, :Wc.shape[1]].set(Wc)
    bc_p = jnp.zeros((1, KC), jnp.float32).at[0, :bc.shape[0]].set(bc)
    nclass = Wc.shape[1]

    def body(h_ref, p_ref, wl_ref, wr_ref, b_ref, g_ref, be_ref,
             batch_ref, wc_ref, bc_ref, out_ref):
        t = jnp.dot(h_ref[...], wl_ref[...], preferred_element_type=jnp.float32)
        t = t + jnp.dot(p_ref[0] + p_ref[1], wr_ref[...],
                        preferred_element_type=jnp.float32)
        t = t + b_ref[...]
        h3 = _bn_relu(t, g_ref[...], be_ref[...])
        bt = batch_ref[...]
        gids = lax.broadcasted_iota(jnp.int32, (G, N), 0)
        oh = (gids == bt).astype(jnp.float32)
        sums = jnp.dot(oh, h3, preferred_element_type=jnp.float32)
        counts = jnp.sum(oh, axis=1, keepdims=True)
        pooled = sums / jnp.maximum(counts, 1.0)
        logits = jnp.dot(pooled, wc_ref[...],
                         preferred_element_type=jnp.float32) + bc_ref[...]
        cids = lax.broadcasted_iota(jnp.int32, (G, KC), 1)
        logits = jnp.where(cids < nclass, logits, -1e30)
        mx = jnp.max(logits, axis=1, keepdims=True)
        ex = jnp.exp(logits - mx)
        out_ref[...] = ex / jnp.sum(ex, axis=1, keepdims=True)

    out = pl.pallas_call(
        body,
        out_shape=jax.ShapeDtypeStruct((G, KC), jnp.float32),
    )(h, parts, Wl, Wr, b.reshape(1, D), g.reshape(1, D), be.reshape(1, D),
      batch.reshape(1, N), Wc_p, bc_p)
    return out[:, :nclass]


def kernel(x, edge_index, edge_attr, batch, W_edge, b_edge, W_nn1, b_nn1,
           g1, be1, W2l, W2r, b2, g2, be2, W3l, W3r, b3, g3, be3, Wc, bc):
    src = edge_index[0]
    dst = edge_index[1]
    G = 64

    e = _tc_edge_mlp(edge_attr, W_edge, b_edge)
    parts1 = _sc_segment_sum(x, src, dst, e, chunk=80)
    h = _tc_stage1(x, parts1, W_nn1, b_nn1, g1, be1)

    parts2 = _sc_segment_sum(h, src, dst, chunk=80)
    h = _tc_stage23(h, parts2, W2l, W2r, b2, g2, be2)

    parts3 = _sc_segment_sum(h, src, dst, chunk=80)
    return _tc_final(h, parts3, W3l, W3r, b3, g3, be3, batch, Wc, bc, G)


# speedup vs baseline: 1.1942x; 1.1942x over previous
import functools

import jax
import jax.numpy as jnp
from jax import lax
from jax.experimental import pallas as pl
from jax.experimental.pallas import tpu as pltpu
from jax.experimental.pallas import tpu_sc as plsc

NC = 2
NS = 16
NW = NC * NS
LANES = 16


def _sc_segment_sum(h, src, dst, e=None, chunk=80, nbuf=4):
    N, D = h.shape
    E = src.shape[0]
    EPW = E // NW
    n_chunks = -(-EPW // chunk)
    n_cpad = 2 * (-(-n_chunks // 2))
    WC = 80 if chunk >= 80 else 40
    NCH = N // WC
    npt = -(-NCH // NS)
    vecs = D // LANES
    fuse = e is not None
    if fuse:
        assert EPW % chunk == 0

    EPP = n_cpad * chunk
    pad = EPP - EPW
    src3 = src.reshape(NW, EPW)
    dst3 = dst.reshape(NW, EPW)
    if pad:
        src3 = jnp.pad(src3, ((0, 0), (0, pad)))
        dst3 = jnp.pad(dst3, ((0, 0), (0, pad)), constant_values=N)
    src3 = src3.reshape(NW * EPP)
    dst3 = dst3.reshape(NW * EPP)

    mesh = plsc.VectorSubcoreMesh(core_axis_name="c", subcore_axis_name="s")

    scratch = [
        pltpu.VMEM_SHARED((N + 8, D), jnp.float32),
        [pltpu.VMEM((chunk,), jnp.int32) for _ in range(2)],
        [pltpu.VMEM((chunk,), jnp.int32) for _ in range(2)],
        [pltpu.VMEM((chunk, D), jnp.float32) for _ in range(2)],
        [pltpu.SemaphoreType.DMA for _ in range(2)],
        [pltpu.SemaphoreType.DMA for _ in range(2)],
        [pltpu.SemaphoreType.DMA for _ in range(2)],
        [pltpu.SemaphoreType.DMA for _ in range(2)],
    ]
    if fuse:
        scratch.append([pltpu.VMEM((chunk, D), jnp.float32) for _ in range(2)])
        scratch.append([pltpu.SemaphoreType.DMA for _ in range(2)])

    def body(*refs):
        if fuse:
            (h_hbm, src_hbm, dst_hbm, e_hbm, out_hbm, acc, srcv, dstv,
             bufs, issems, idsems, gsems, ssems, ebufs, esems) = refs
        else:
            (h_hbm, src_hbm, dst_hbm, out_hbm, acc, srcv, dstv,
             bufs, issems, idsems, gsems, ssems) = refs

        c = lax.axis_index("c")
        s = lax.axis_index("s")
        wid = s * NC + c

        zero = jnp.zeros((LANES,), jnp.float32)

        def zrow(r, _):
            for cc in range(vecs):
                bufs[0][r, pl.ds(cc * LANES, LANES)] = zero
            return 0

        lax.fori_loop(0, WC, zrow, 0, unroll=False)

        for j in range(npt):
            k = s + j * NS

            @pl.when(k < NCH)
            def _():
                pltpu.sync_copy(bufs[0].at[pl.ds(0, WC), :],
                                acc.at[pl.ds(k * WC, WC), :])

        plsc.subcore_barrier()

        def chunkfn(k, _):
            ebase = wid * EPP + k * chunk
            pltpu.sync_copy(src_hbm.at[pl.ds(ebase, chunk)], srcv[0])
            pltpu.sync_copy(dst_hbm.at[pl.ds(ebase, chunk)], dstv[0])
            pltpu.sync_copy(h_hbm.at[srcv[0]], bufs[0])
            if fuse:
                kc = jnp.minimum(k, n_chunks - 1)
                base = wid * EPW + kc * chunk
                pltpu.sync_copy(e_hbm.at[pl.ds(base, chunk), :], ebufs[0])

                def rowfn(r, _):
                    for cc in range(vecs):
                        sl = pl.ds(cc * LANES, LANES)
                        v = bufs[0][r, sl] + ebufs[0][r, sl]
                        bufs[0][r, sl] = jnp.maximum(v, 0.0)
                    return 0

                lax.fori_loop(0, chunk, rowfn, 0, unroll=False)
            pltpu.sync_copy(bufs[0], acc.at[dstv[0]], add=True)
            return 0

        lax.fori_loop(0, n_cpad, chunkfn, 0, unroll=False)
        plsc.subcore_barrier()

        for j in range(npt):
            k = s + j * NS

            @pl.when(k < NCH)
            def _():
                pltpu.sync_copy(acc.at[pl.ds(k * WC, WC), :],
                                bufs[0].at[pl.ds(0, WC), :])
                pltpu.sync_copy(bufs[0].at[pl.ds(0, WC), :],
                                out_hbm.at[c, pl.ds(k * WC, WC), :])

    kern = pl.kernel(
        body,
        out_type=jax.ShapeDtypeStruct((2, N, D), jnp.float32),
        mesh=mesh,
        scratch_types=scratch,
    )
    if fuse:
        return kern(h, src3, dst3, e)
    return kern(h, src3, dst3)


def _tc_edge_mlp(edge_attr, W_edge, b_edge):
    E, DE = edge_attr.shape
    D = W_edge.shape[1]
    BE = 4000

    def body(ea_ref, w_ref, b_ref, out_ref):
        out_ref[...] = jnp.dot(ea_ref[...], w_ref[...],
                               preferred_element_type=jnp.float32) + b_ref[...]

    return pl.pallas_call(
        body,
        grid=(E // BE,),
        in_specs=[
            pl.BlockSpec((BE, DE), lambda i: (i, 0)),
            pl.BlockSpec((DE, D), lambda i: (0, 0)),
            pl.BlockSpec((1, D), lambda i: (0, 0)),
        ],
        out_specs=pl.BlockSpec((BE, D), lambda i: (i, 0)),
        out_shape=jax.ShapeDtypeStruct((E, D), jnp.float32),
    )(edge_attr, W_edge, b_edge.reshape(1, D))


def _bn_relu(t, g, be):
    m = jnp.mean(t, axis=0, keepdims=True)
    d = t - m
    v = jnp.mean(d * d, axis=0, keepdims=True)
    return jnp.maximum(g * d * lax.rsqrt(v + 1e-5) + be, 0.0)


def _tc_stage1(x, parts, W, b, g, be):
    N, D = x.shape

    def body(x_ref, p_ref, w_ref, b_ref, g_ref, be_ref, out_ref):
        a = x_ref[...] + p_ref[0] + p_ref[1]
        t = jnp.dot(a, w_ref[...], preferred_element_type=jnp.float32)
        t = t + b_ref[...]
        out_ref[...] = _bn_relu(t, g_ref[...], be_ref[...])

    return pl.pallas_call(
        body,
        out_shape=jax.ShapeDtypeStruct((N, D), jnp.float32),
    )(x, parts, W, b.reshape(1, D), g.reshape(1, D), be.reshape(1, D))


def _tc_stage23(h, parts, Wl, Wr, b, g, be):
    N, D = h.shape

    def body(h_ref, p_ref, wl_ref, wr_ref, b_ref, g_ref, be_ref, out_ref):
        t = jnp.dot(h_ref[...], wl_ref[...], preferred_element_type=jnp.float32)
        t = t + jnp.dot(p_ref[0] + p_ref[1], wr_ref[...],
                        preferred_element_type=jnp.float32)
        t = t + b_ref[...]
        out_ref[...] = _bn_relu(t, g_ref[...], be_ref[...])

    return pl.pallas_call(
        body,
        out_shape=jax.ShapeDtypeStruct((N, D), jnp.float32),
    )(h, parts, Wl, Wr, b.reshape(1, D), g.reshape(1, D), be.reshape(1, D))


def _tc_final(h, parts, Wl, Wr, b, g, be, batch, Wc, bc, G):
    N, D = h.shape
    KC = 8
    Wc_p = jnp.zeros((D, KC), jnp.float32).at[:, :Wc.shape[1]].set(Wc)
    bc_p = jnp.zeros((1, KC), jnp.float32).at[0, :bc.shape[0]].set(bc)
    nclass = Wc.shape[1]

    def body(h_ref, p_ref, wl_ref, wr_ref, b_ref, g_ref, be_ref,
             batch_ref, wc_ref, bc_ref, out_ref):
        t = jnp.dot(h_ref[...], wl_ref[...], preferred_element_type=jnp.float32)
        t = t + jnp.dot(p_ref[0] + p_ref[1], wr_ref[...],
                        preferred_element_type=jnp.float32)
        t = t + b_ref[...]
        h3 = _bn_relu(t, g_ref[...], be_ref[...])
        bt = batch_ref[...]
        gids = lax.broadcasted_iota(jnp.int32, (G, N), 0)
        oh = (gids == bt).astype(jnp.float32)
        sums = jnp.dot(oh, h3, preferred_element_type=jnp.float32)
        counts = jnp.sum(oh, axis=1, keepdims=True)
        pooled = sums / jnp.maximum(counts, 1.0)
        logits = jnp.dot(pooled, wc_ref[...],
                         preferred_element_type=jnp.float32) + bc_ref[...]
        cids = lax.broadcasted_iota(jnp.int32, (G, KC), 1)
        logits = jnp.where(cids < nclass, logits, -1e30)
        mx = jnp.max(logits, axis=1, keepdims=True)
        ex = jnp.exp(logits - mx)
        out_ref[...] = ex / jnp.sum(ex, axis=1, keepdims=True)

    out = pl.pallas_call(
        body,
        out_shape=jax.ShapeDtypeStruct((G, KC), jnp.float32),
    )(h, parts, Wl, Wr, b.reshape(1, D), g.reshape(1, D), be.reshape(1, D),
      batch.reshape(1, N), Wc_p, bc_p)
    return out[:, :nclass]


def kernel(x, edge_index, edge_attr, batch, W_edge, b_edge, W_nn1, b_nn1,
           g1, be1, W2l, W2r, b2, g2, be2, W3l, W3r, b3, g3, be3, Wc, bc):
    src = edge_index[0]
    dst = edge_index[1]
    G = 64

    e = _tc_edge_mlp(edge_attr, W_edge, b_edge)
    parts1 = _sc_segment_sum(x, src, dst, e, chunk=80)
    h = _tc_stage1(x, parts1, W_nn1, b_nn1, g1, be1)

    parts2 = _sc_segment_sum(h, src, dst, chunk=80)
    h = _tc_stage23(h, parts2, W2l, W2r, b2, g2, be2)

    parts3 = _sc_segment_sum(h, src, dst, chunk=80)
    return _tc_final(h, parts3, W3l, W3r, b3, g3, be3, batch, Wc, bc, G)


# speedup vs baseline: 1.7255x; 1.4449x over previous
import functools

import jax
import jax.numpy as jnp
from jax import lax
from jax.experimental import pallas as pl
from jax.experimental.pallas import tpu as pltpu
from jax.experimental.pallas import tpu_sc as plsc

NC = 2
NS = 16
NW = NC * NS
LANES = 16


def _sc_segment_sum(h, src, dst, e=None, chunk=80, nbuf=4):
    N, D = h.shape
    E = src.shape[0]
    EPW = E // NW
    n_chunks = -(-EPW // chunk)
    n_cpad = 2 * (-(-n_chunks // 2))
    WC = 80 if chunk >= 80 else 40
    NCH = N // WC
    npt = -(-NCH // NS)
    vecs = D // LANES
    fuse = e is not None
    if fuse:
        assert EPW % chunk == 0

    EPP = n_cpad * chunk
    pad = EPP - EPW
    src3 = src.reshape(NW, EPW)
    dst3 = dst.reshape(NW, EPW)
    if pad:
        src3 = jnp.pad(src3, ((0, 0), (0, pad)))
        dst3 = jnp.pad(dst3, ((0, 0), (0, pad)), constant_values=N)
    src3 = src3.reshape(NW * EPP)
    dst3 = dst3.reshape(NW * EPP)

    mesh = plsc.VectorSubcoreMesh(core_axis_name="c", subcore_axis_name="s")

    scratch = [
        pltpu.VMEM_SHARED((N + 8, D), jnp.float32),
        [pltpu.VMEM((chunk,), jnp.int32) for _ in range(2)],
        [pltpu.VMEM((chunk,), jnp.int32) for _ in range(2)],
        [pltpu.VMEM((chunk, D), jnp.float32) for _ in range(2)],
        [pltpu.SemaphoreType.DMA for _ in range(2)],
        [pltpu.SemaphoreType.DMA for _ in range(2)],
        [pltpu.SemaphoreType.DMA for _ in range(2)],
        [pltpu.SemaphoreType.DMA for _ in range(2)],
    ]
    if fuse:
        scratch.append([pltpu.VMEM((chunk, D), jnp.float32) for _ in range(2)])
        scratch.append([pltpu.SemaphoreType.DMA for _ in range(2)])

    def body(*refs):
        if fuse:
            (h_hbm, src_hbm, dst_hbm, e_hbm, out_hbm, acc, srcv, dstv,
             bufs, issems, idsems, gsems, ssems, ebufs, esems) = refs
        else:
            (h_hbm, src_hbm, dst_hbm, out_hbm, acc, srcv, dstv,
             bufs, issems, idsems, gsems, ssems) = refs

        c = lax.axis_index("c")
        s = lax.axis_index("s")
        wid = s * NC + c

        zero = jnp.zeros((LANES,), jnp.float32)

        def zrow(r, _):
            for cc in range(vecs):
                bufs[0][r, pl.ds(cc * LANES, LANES)] = zero
            return 0

        lax.fori_loop(0, WC, zrow, 0, unroll=False)

        for j in range(npt):
            k = s + j * NS

            @pl.when(k < NCH)
            def _():
                pltpu.sync_copy(bufs[0].at[pl.ds(0, WC), :],
                                acc.at[pl.ds(k * WC, WC), :])

        plsc.subcore_barrier()

        def pairfn(j, _):
            ii, dd, ee = [], [], []
            for b in range(2):
                k = j * 2 + b
                ebase = wid * EPP + k * chunk
                ii.append(pltpu.async_copy(
                    src_hbm.at[pl.ds(ebase, chunk)], srcv[b], issems[b]))
                dd.append(pltpu.async_copy(
                    dst_hbm.at[pl.ds(ebase, chunk)], dstv[b], idsems[b]))
                if fuse:
                    kc = jnp.minimum(k, n_chunks - 1)
                    base = wid * EPW + kc * chunk
                    ee.append(pltpu.async_copy(
                        e_hbm.at[pl.ds(base, chunk), :], ebufs[b], esems[b]))
            gg = []
            for b in range(2):
                ii[b].wait()
                gg.append(pltpu.async_copy(h_hbm.at[srcv[b]], bufs[b],
                                           gsems[b]))
            ss = []
            for b in range(2):
                gg[b].wait()
                if fuse:
                    ee[b].wait()

                    def rowfn(r, _):
                        for cc in range(vecs):
                            sl = pl.ds(cc * LANES, LANES)
                            v = bufs[b][r, sl] + ebufs[b][r, sl]
                            bufs[b][r, sl] = jnp.maximum(v, 0.0)
                        return 0

                    lax.fori_loop(0, chunk, rowfn, 0, unroll=False)
                dd[b].wait()
                ss.append(pltpu.async_copy(bufs[b], acc.at[dstv[b]],
                                           ssems[b], add=True))
            ss[0].wait()
            ss[1].wait()
            return 0

        lax.fori_loop(0, n_cpad // 2, pairfn, 0, unroll=False)
        plsc.subcore_barrier()

        for j in range(npt):
            k = s + j * NS

            @pl.when(k < NCH)
            def _():
                pltpu.sync_copy(acc.at[pl.ds(k * WC, WC), :],
                                bufs[0].at[pl.ds(0, WC), :])
                pltpu.sync_copy(bufs[0].at[pl.ds(0, WC), :],
                                out_hbm.at[c, pl.ds(k * WC, WC), :])

    kern = pl.kernel(
        body,
        out_type=jax.ShapeDtypeStruct((2, N, D), jnp.float32),
        mesh=mesh,
        scratch_types=scratch,
    )
    if fuse:
        return kern(h, src3, dst3, e)
    return kern(h, src3, dst3)


def _tc_edge_mlp(edge_attr, W_edge, b_edge):
    E, DE = edge_attr.shape
    D = W_edge.shape[1]
    BE = 4000

    def body(ea_ref, w_ref, b_ref, out_ref):
        out_ref[...] = jnp.dot(ea_ref[...], w_ref[...],
                               preferred_element_type=jnp.float32) + b_ref[...]

    return pl.pallas_call(
        body,
        grid=(E // BE,),
        in_specs=[
            pl.BlockSpec((BE, DE), lambda i: (i, 0)),
            pl.BlockSpec((DE, D), lambda i: (0, 0)),
            pl.BlockSpec((1, D), lambda i: (0, 0)),
        ],
        out_specs=pl.BlockSpec((BE, D), lambda i: (i, 0)),
        out_shape=jax.ShapeDtypeStruct((E, D), jnp.float32),
    )(edge_attr, W_edge, b_edge.reshape(1, D))


def _bn_relu(t, g, be):
    m = jnp.mean(t, axis=0, keepdims=True)
    d = t - m
    v = jnp.mean(d * d, axis=0, keepdims=True)
    return jnp.maximum(g * d * lax.rsqrt(v + 1e-5) + be, 0.0)


def _tc_stage1(x, parts, W, b, g, be):
    N, D = x.shape

    def body(x_ref, p_ref, w_ref, b_ref, g_ref, be_ref, out_ref):
        a = x_ref[...] + p_ref[0] + p_ref[1]
        t = jnp.dot(a, w_ref[...], preferred_element_type=jnp.float32)
        t = t + b_ref[...]
        out_ref[...] = _bn_relu(t, g_ref[...], be_ref[...])

    return pl.pallas_call(
        body,
        out_shape=jax.ShapeDtypeStruct((N, D), jnp.float32),
    )(x, parts, W, b.reshape(1, D), g.reshape(1, D), be.reshape(1, D))


def _tc_stage23(h, parts, Wl, Wr, b, g, be):
    N, D = h.shape

    def body(h_ref, p_ref, wl_ref, wr_ref, b_ref, g_ref, be_ref, out_ref):
        t = jnp.dot(h_ref[...], wl_ref[...], preferred_element_type=jnp.float32)
        t = t + jnp.dot(p_ref[0] + p_ref[1], wr_ref[...],
                        preferred_element_type=jnp.float32)
        t = t + b_ref[...]
        out_ref[...] = _bn_relu(t, g_ref[...], be_ref[...])

    return pl.pallas_call(
        body,
        out_shape=jax.ShapeDtypeStruct((N, D), jnp.float32),
    )(h, parts, Wl, Wr, b.reshape(1, D), g.reshape(1, D), be.reshape(1, D))


def _tc_final(h, parts, Wl, Wr, b, g, be, batch, Wc, bc, G):
    N, D = h.shape
    KC = 8
    Wc_p = jnp.zeros((D, KC), jnp.float32).at[:, :Wc.shape[1]].set(Wc)
    bc_p = jnp.zeros((1, KC), jnp.float32).at[0, :bc.shape[0]].set(bc)
    nclass = Wc.shape[1]

    def body(h_ref, p_ref, wl_ref, wr_ref, b_ref, g_ref, be_ref,
             batch_ref, wc_ref, bc_ref, out_ref):
        t = jnp.dot(h_ref[...], wl_ref[...], preferred_element_type=jnp.float32)
        t = t + jnp.dot(p_ref[0] + p_ref[1], wr_ref[...],
                        preferred_element_type=jnp.float32)
        t = t + b_ref[...]
        h3 = _bn_relu(t, g_ref[...], be_ref[...])
        bt = batch_ref[...]
        gids = lax.broadcasted_iota(jnp.int32, (G, N), 0)
        oh = (gids == bt).astype(jnp.float32)
        sums = jnp.dot(oh, h3, preferred_element_type=jnp.float32)
        counts = jnp.sum(oh, axis=1, keepdims=True)
        pooled = sums / jnp.maximum(counts, 1.0)
        logits = jnp.dot(pooled, wc_ref[...],
                         preferred_element_type=jnp.float32) + bc_ref[...]
        cids = lax.broadcasted_iota(jnp.int32, (G, KC), 1)
        logits = jnp.where(cids < nclass, logits, -1e30)
        mx = jnp.max(logits, axis=1, keepdims=True)
        ex = jnp.exp(logits - mx)
        out_ref[...] = ex / jnp.sum(ex, axis=1, keepdims=True)

    out = pl.pallas_call(
        body,
        out_shape=jax.ShapeDtypeStruct((G, KC), jnp.float32),
    )(h, parts, Wl, Wr, b.reshape(1, D), g.reshape(1, D), be.reshape(1, D),
      batch.reshape(1, N), Wc_p, bc_p)
    return out[:, :nclass]


def kernel(x, edge_index, edge_attr, batch, W_edge, b_edge, W_nn1, b_nn1,
           g1, be1, W2l, W2r, b2, g2, be2, W3l, W3r, b3, g3, be3, Wc, bc):
    src = edge_index[0]
    dst = edge_index[1]
    G = 64

    e = _tc_edge_mlp(edge_attr, W_edge, b_edge)
    parts1 = _sc_segment_sum(x, src, dst, e, chunk=80)
    h = _tc_stage1(x, parts1, W_nn1, b_nn1, g1, be1)

    parts2 = _sc_segment_sum(h, src, dst, chunk=120)
    h = _tc_stage23(h, parts2, W2l, W2r, b2, g2, be2)

    parts3 = _sc_segment_sum(h, src, dst, chunk=120)
    return _tc_final(h, parts3, W3l, W3r, b3, g3, be3, batch, Wc, bc, G)
